# Initial kernel scaffold; baseline (speedup 1.0000x reference)
import jax
import jax.numpy as jnp
from jax.experimental import pallas as pl


def kernel(region_mouth, region_left_eye, region_right_eye, audio_embeddings, W_mouth, b_mouth, W_left_eye, b_left_eye, W_right_eye, b_right_eye, W_audio, b_audio, gW0, gas0, gad0, gb0, gW1, gas1, gad1, gb1, gW2, gas2, gad2, gb2, ln_g, ln_b):
    raise NotImplementedError("write your pallas kernel here")


# speedup vs baseline: 175.7551x; 175.7551x over previous
import jax
import jax.numpy as jnp
from jax.experimental import pallas as pl

_HID = 256
_F32 = jnp.float32


def _dot(a, b):
    return jnp.dot(a, b, preferred_element_type=_F32)


def _prep_body(gW0, gW1, gW2, gb0, gb1, gb2, Wm, Wl, Wr, Wa, bm, bl, br, ba,
               W0s, b0s, Fs, cs):
    W12 = _dot(gW1[...], gW2[...])
    W012 = _dot(gW0[...], W12)
    d = _dot(_dot(gb0[...], gW1[...]) + gb1[...], gW2[...]) + gb2[...]
    W0s[0, :, :] = _dot(Wm[...], gW0[...])
    W0s[1, :, :] = _dot(Wa[...], gW0[...])
    b0s[0, :, :] = _dot(bm[...], gW0[...])
    b0s[1, :, :] = _dot(ba[...], gW0[...])
    ins = ((Wm, bm), (Wl, bl), (Wr, br), (Wa, ba))
    for g, (W_in, b_in) in enumerate(ins):
        Fs[g, :, :] = _dot(W_in[...], W012)
        cs[g, :, :] = _dot(b_in[...], W012) + d


def _leaky(z):
    return jnp.where(z > 0, z, 0.2 * z)


def _stencil(h, hp_last, ls, ld, lsp_last, valid, gb):
    neg = jnp.float32(-1e30)
    outs = []
    for r in (0, 1, 2):
        o1, o2 = [q for q in (0, 1, 2) if q != r]
        dr = ld[r]
        e1 = _leaky(ls[o1] + dr)
        e2 = _leaky(ls[o2] + dr)
        es = _leaky(ls[r] + dr)
        ls_prev = jnp.concatenate([lsp_last[r], ls[r][:-1]], axis=0)
        et = jnp.where(valid, _leaky(ls_prev + dr), neg)
        m = jnp.maximum(jnp.maximum(e1, e2), jnp.maximum(es, et))
        w1 = jnp.exp(e1 - m)
        w2 = jnp.exp(e2 - m)
        wsf = jnp.exp(es - m)
        wt = jnp.where(valid, jnp.exp(et - m), 0.0)
        h_prev = jnp.concatenate([hp_last[r], h[r][:-1]], axis=0)
        num = w1 * h[o1] + w2 * h[o2] + wsf * h[r] + wt * h_prev
        den = w1 + w2 + wsf + wt + 1e-16
        outs.append(num / den + gb)
    da = ld[3]
    e1 = _leaky(ls[1] + da)
    e2 = _leaky(ls[2] + da)
    es = _leaky(ls[3] + da)
    m = jnp.maximum(jnp.maximum(e1, e2), es)
    w1 = jnp.exp(e1 - m)
    w2 = jnp.exp(e2 - m)
    wsf = jnp.exp(es - m)
    num = w1 * h[1] + w2 * h[2] + wsf * h[3]
    den = w1 + w2 + wsf + 1e-16
    outs.append(num / den + gb)
    return outs


def _row_logit(h, v):
    return jnp.sum(h * v, axis=1, keepdims=True)


def _layer0_body(xc_ref, xp_ref, W0s_ref, b0s_ref, as_ref, ad_ref, gb_ref,
                 o_ref):
    BT = xc_ref.shape[1]
    Wm, Wa = W0s_ref[0], W0s_ref[1]
    bm, ba = b0s_ref[0], b0s_ref[1]
    asv, adv = as_ref[...], ad_ref[...]
    h = [_dot(xc_ref[r], Wm) + bm for r in range(3)]
    h.append(_dot(xc_ref[3], Wa) + ba)
    hp_last = [_dot(xp_ref[r, BT - 1:BT, :], Wm) + bm for r in range(3)]
    ls = [_row_logit(h[r], asv) for r in range(4)]
    ld = [_row_logit(h[r], adv) for r in range(4)]
    lsp_last = [_row_logit(hp_last[r], asv) for r in range(3)]
    tloc = jax.lax.broadcasted_iota(jnp.int32, (BT, 1), 0)
    valid = (pl.program_id(0) * BT + tloc) >= 1
    outs = _stencil(h, hp_last, ls, ld, lsp_last, valid, gb_ref[...])
    for r in range(4):
        o_ref[r, :, :] = outs[r]


def _layer_mid_body(xc_ref, xp_ref, W_ref, as_ref, ad_ref, gb_ref, o_ref):
    BT = xc_ref.shape[1]
    W = W_ref[...]
    asv, adv = as_ref[...], ad_ref[...]
    h = [_dot(xc_ref[r], W) for r in range(4)]
    hp_last = [_dot(xp_ref[r, BT - 1:BT, :], W) for r in range(3)]
    ls = [_row_logit(h[r], asv) for r in range(4)]
    ld = [_row_logit(h[r], adv) for r in range(4)]
    lsp_last = [_row_logit(hp_last[r], asv) for r in range(3)]
    tloc = jax.lax.broadcasted_iota(jnp.int32, (BT, 1), 0)
    valid = (pl.program_id(0) * BT + tloc) >= 1
    outs = _stencil(h, hp_last, ls, ld, lsp_last, valid, gb_ref[...])
    for r in range(4):
        o_ref[r, :, :] = outs[r]


def _ln_rowsum(y, g, b):
    mu = jnp.mean(y, axis=1, keepdims=True)
    yc = y - mu
    var = jnp.mean(yc * yc, axis=1, keepdims=True)
    z = yc / jnp.sqrt(var + 1e-5) * g + b
    return jnp.sum(z, axis=0, keepdims=True)


def _layer_last_body(xc_ref, xp_ref, W_ref, as_ref, ad_ref, gb_ref,
                     lng_ref, lnb_ref, o_ref):
    BT = xc_ref.shape[1]
    W = W_ref[...]
    asv, adv = as_ref[...], ad_ref[...]
    h = [_dot(xc_ref[r], W) for r in range(4)]
    hp_last = [_dot(xp_ref[r, BT - 1:BT, :], W) for r in range(3)]
    ls = [_row_logit(h[r], asv) for r in range(4)]
    ld = [_row_logit(h[r], adv) for r in range(4)]
    lsp_last = [_row_logit(hp_last[r], asv) for r in range(3)]
    tloc = jax.lax.broadcasted_iota(jnp.int32, (BT, 1), 0)
    valid = (pl.program_id(0) * BT + tloc) >= 1
    outs = _stencil(h, hp_last, ls, ld, lsp_last, valid, gb_ref[...])
    lng, lnb = lng_ref[...], lnb_ref[...]
    s = _ln_rowsum(outs[0], lng, lnb)
    for r in range(1, 4):
        s = s + _ln_rowsum(outs[r], lng, lnb)

    @pl.when(pl.program_id(0) == 0)
    def _init():
        o_ref[...] = jnp.zeros_like(o_ref)

    o_ref[...] += s


def _passive_body(x_ref, F_ref, c_ref, g_ref, b_ref, o_ref):
    y = _dot(x_ref[...], F_ref[0]) + c_ref[0]
    s = _ln_rowsum(y, g_ref[...], b_ref[...])

    @pl.when(pl.program_id(0) == 0)
    def _init():
        o_ref[...] = jnp.zeros_like(o_ref)

    o_ref[...] += s


def kernel(region_mouth, region_left_eye, region_right_eye, audio_embeddings,
           W_mouth, b_mouth, W_left_eye, b_left_eye, W_right_eye, b_right_eye,
           W_audio, b_audio, gW0, gas0, gad0, gb0, gW1, gas1, gad1, gb1,
           gW2, gas2, gad2, gb2, ln_g, ln_b):
    B, T, D = region_mouth.shape
    T_a, A = audio_embeddings.shape[1], audio_embeddings.shape[2]
    N_total = 3 * B * T + B * T_a
    f32 = _F32

    r2 = lambda v: v.reshape(1, _HID)
    bm, bl, br, ba = r2(b_mouth), r2(b_left_eye), r2(b_right_eye), r2(b_audio)
    gas = [g.reshape(1, _HID) for g in (gas0, gas1, gas2)]
    gad = [g.reshape(1, _HID) for g in (gad0, gad1, gad2)]
    gbr = [r2(gb0), r2(gb1), r2(gb2)]
    lng, lnb = r2(ln_g), r2(ln_b)

    W0s, b0s, Fs, cs = pl.pallas_call(
        _prep_body,
        out_shape=[
            jax.ShapeDtypeStruct((2, D, _HID), f32),
            jax.ShapeDtypeStruct((2, 1, _HID), f32),
            jax.ShapeDtypeStruct((4, D, _HID), f32),
            jax.ShapeDtypeStruct((4, 1, _HID), f32),
        ],
    )(gW0, gW1, gW2, gbr[0], gbr[1], gbr[2],
      W_mouth, W_left_eye, W_right_eye, W_audio, bm, bl, br, ba)

    raw_active = jnp.concatenate(
        [region_mouth[:3], audio_embeddings[0:1]], axis=0)

    BT = 1024
    NT = T // BT
    vec_bs = pl.BlockSpec((1, _HID), lambda i: (0, 0))

    def tile_bs(depth):
        return pl.BlockSpec((4, BT, depth), lambda i: (0, i, 0))

    def halo_bs(depth):
        return pl.BlockSpec((4, BT, depth),
                            lambda i: (0, jnp.maximum(i - 1, 0), 0))

    x = pl.pallas_call(
        _layer0_body,
        grid=(NT,),
        in_specs=[
            tile_bs(D), halo_bs(D),
            pl.BlockSpec((2, D, _HID), lambda i: (0, 0, 0)),
            pl.BlockSpec((2, 1, _HID), lambda i: (0, 0, 0)),
            vec_bs, vec_bs, vec_bs,
        ],
        out_specs=tile_bs(_HID),
        out_shape=jax.ShapeDtypeStruct((4, T, _HID), f32),
    )(raw_active, raw_active, W0s, b0s, gas[0], gad[0], gbr[0])

    x = pl.pallas_call(
        _layer_mid_body,
        grid=(NT,),
        in_specs=[
            tile_bs(_HID), halo_bs(_HID),
            pl.BlockSpec((_HID, _HID), lambda i: (0, 0)),
            vec_bs, vec_bs, vec_bs,
        ],
        out_specs=tile_bs(_HID),
        out_shape=jax.ShapeDtypeStruct((4, T, _HID), f32),
    )(x, x, gW1, gas[1], gad[1], gbr[1])

    s_active = pl.pallas_call(
        _layer_last_body,
        grid=(NT,),
        in_specs=[
            tile_bs(_HID), halo_bs(_HID),
            pl.BlockSpec((_HID, _HID), lambda i: (0, 0)),
            vec_bs, vec_bs, vec_bs, vec_bs, vec_bs,
        ],
        out_specs=pl.BlockSpec((1, _HID), lambda i: (0, 0)),
        out_shape=jax.ShapeDtypeStruct((1, _HID), f32),
    )(x, x, gW2, gas[2], gad[2], gbr[2], lng, lnb)

    def passive_sum(raw, group):
        n = raw.shape[0]
        tiles = n // T
        return pl.pallas_call(
            _passive_body,
            grid=(tiles,),
            in_specs=[
                pl.BlockSpec((T, D), lambda i: (i, 0)),
                pl.BlockSpec((1, D, _HID), lambda i, g=group: (g, 0, 0)),
                pl.BlockSpec((1, 1, _HID), lambda i, g=group: (g, 0, 0)),
                vec_bs, vec_bs,
            ],
            out_specs=pl.BlockSpec((1, _HID), lambda i: (0, 0)),
            out_shape=jax.ShapeDtypeStruct((1, _HID), f32),
        )(raw, Fs, cs, lng, lnb)

    s_m = passive_sum(region_mouth[3], 0)
    s_l = passive_sum(region_left_eye.reshape(B * T, D), 1)
    s_r = passive_sum(region_right_eye.reshape(B * T, D), 2)
    s_a = passive_sum(audio_embeddings[1:].reshape((B - 1) * T_a, A), 3)

    total = s_active + s_m + s_l + s_r + s_a
    return total / jnp.float32(N_total)


# speedup vs baseline: 204.1890x; 1.1618x over previous
import jax
import jax.numpy as jnp
from jax.experimental import pallas as pl
from jax.experimental.pallas import tpu as pltpu

_HID = 256
_F32 = jnp.float32


def _dot(a, b):
    return jnp.dot(a, b, preferred_element_type=_F32)


def _prep_body(gW0, gW1, gW2, gb0, gb1, gb2, Wm, Wl, Wr, Wa, bm, bl, br, ba,
               W0s, b0s, Fs, cs):
    W12 = _dot(gW1[...], gW2[...])
    W012 = _dot(gW0[...], W12)
    d = _dot(_dot(gb0[...], gW1[...]) + gb1[...], gW2[...]) + gb2[...]
    W0s[0, :, :] = _dot(Wm[...], gW0[...])
    W0s[1, :, :] = _dot(Wa[...], gW0[...])
    b0s[0, :, :] = _dot(bm[...], gW0[...])
    b0s[1, :, :] = _dot(ba[...], gW0[...])
    ins = ((Wm, bm), (Wl, bl), (Wr, br), (Wa, ba))
    for g, (W_in, b_in) in enumerate(ins):
        Fs[g, :, :] = _dot(W_in[...], W012)
        cs[g, :, :] = _dot(b_in[...], W012) + d


def _leaky(z):
    return jnp.where(z > 0, z, 0.2 * z)


def _stencil(h, hp_last, ls, ld, lsp_last, valid, gb):
    neg = jnp.float32(-1e30)
    outs = []
    for r in (0, 1, 2):
        o1, o2 = [q for q in (0, 1, 2) if q != r]
        dr = ld[r]
        e1 = _leaky(ls[o1] + dr)
        e2 = _leaky(ls[o2] + dr)
        es = _leaky(ls[r] + dr)
        ls_prev = jnp.concatenate([lsp_last[r], ls[r][:-1]], axis=0)
        et = jnp.where(valid, _leaky(ls_prev + dr), neg)
        m = jnp.maximum(jnp.maximum(e1, e2), jnp.maximum(es, et))
        w1 = jnp.exp(e1 - m)
        w2 = jnp.exp(e2 - m)
        wsf = jnp.exp(es - m)
        wt = jnp.where(valid, jnp.exp(et - m), 0.0)
        h_prev = jnp.concatenate([hp_last[r], h[r][:-1]], axis=0)
        num = w1 * h[o1] + w2 * h[o2] + wsf * h[r] + wt * h_prev
        den = w1 + w2 + wsf + wt + 1e-16
        outs.append(num / den + gb)
    da = ld[3]
    e1 = _leaky(ls[1] + da)
    e2 = _leaky(ls[2] + da)
    es = _leaky(ls[3] + da)
    m = jnp.maximum(jnp.maximum(e1, e2), es)
    w1 = jnp.exp(e1 - m)
    w2 = jnp.exp(e2 - m)
    wsf = jnp.exp(es - m)
    num = w1 * h[1] + w2 * h[2] + wsf * h[3]
    den = w1 + w2 + wsf + 1e-16
    outs.append(num / den + gb)
    return outs


def _row_logit(h, v):
    return jnp.sum(h * v, axis=1, keepdims=True)


def _ln_rowsum(y, g, b):
    mu = jnp.mean(y, axis=1, keepdims=True)
    yc = y - mu
    var = jnp.mean(yc * yc, axis=1, keepdims=True)
    z = yc / jnp.sqrt(var + 1e-5) * g + b
    return jnp.sum(z, axis=0, keepdims=True)


def _active_body(xm_ref, xa_ref, W0s_ref, b0s_ref, gW1_ref, gW2_ref,
                 as0_ref, ad0_ref, as1_ref, ad1_ref, as2_ref, ad2_ref,
                 gb0_ref, gb1_ref, gb2_ref, lng_ref, lnb_ref,
                 o_ref, c0_ref, c1_ref, c2_ref):
    BT = xm_ref.shape[1]
    tloc = jax.lax.broadcasted_iota(jnp.int32, (BT, 1), 0)
    valid = (pl.program_id(0) * BT + tloc) >= 1

    @pl.when(pl.program_id(0) == 0)
    def _init():
        c0_ref[...] = jnp.zeros_like(c0_ref)
        c1_ref[...] = jnp.zeros_like(c1_ref)
        c2_ref[...] = jnp.zeros_like(c2_ref)
        o_ref[...] = jnp.zeros_like(o_ref)

    def run_layer(h, c_ref, as_ref, ad_ref, gb_ref):
        hp_last = [c_ref[r:r + 1, :] for r in range(3)]
        asv, adv = as_ref[...], ad_ref[...]
        ls = [_row_logit(h[r], asv) for r in range(4)]
        ld = [_row_logit(h[r], adv) for r in range(4)]
        lsp_last = [_row_logit(hp_last[r], asv) for r in range(3)]
        outs = _stencil(h, hp_last, ls, ld, lsp_last, valid, gb_ref[...])
        for r in range(3):
            c_ref[r:r + 1, :] = h[r][BT - 1:BT, :]
        return outs

    Wm, Wa = W0s_ref[0], W0s_ref[1]
    h0 = [_dot(xm_ref[r], Wm) + b0s_ref[0] for r in range(3)]
    h0.append(_dot(xa_ref[0], Wa) + b0s_ref[1])
    x1 = run_layer(h0, c0_ref, as0_ref, ad0_ref, gb0_ref)

    W1 = gW1_ref[...]
    h1 = [_dot(x1[r], W1) for r in range(4)]
    x2 = run_layer(h1, c1_ref, as1_ref, ad1_ref, gb1_ref)

    W2 = gW2_ref[...]
    h2 = [_dot(x2[r], W2) for r in range(4)]
    x3 = run_layer(h2, c2_ref, as2_ref, ad2_ref, gb2_ref)
    lng, lnb = lng_ref[...], lnb_ref[...]
    s = _ln_rowsum(x3[0], lng, lnb)
    for r in range(1, 4):
        s = s + _ln_rowsum(x3[r], lng, lnb)
    o_ref[...] += s


def _passive_body(x_ref, F_ref, c_ref, g_ref, b_ref, o_ref):
    y = _dot(x_ref[...], F_ref[0]) + c_ref[0]
    s = _ln_rowsum(y, g_ref[...], b_ref[...])

    @pl.when(pl.program_id(0) == 0)
    def _init():
        o_ref[...] = jnp.zeros_like(o_ref)

    o_ref[...] += s


def kernel(region_mouth, region_left_eye, region_right_eye, audio_embeddings,
           W_mouth, b_mouth, W_left_eye, b_left_eye, W_right_eye, b_right_eye,
           W_audio, b_audio, gW0, gas0, gad0, gb0, gW1, gas1, gad1, gb1,
           gW2, gas2, gad2, gb2, ln_g, ln_b):
    B, T, D = region_mouth.shape
    T_a, A = audio_embeddings.shape[1], audio_embeddings.shape[2]
    N_total = 3 * B * T + B * T_a
    f32 = _F32

    r2 = lambda v: v.reshape(1, _HID)
    bm, bl, br, ba = r2(b_mouth), r2(b_left_eye), r2(b_right_eye), r2(b_audio)
    gas = [g.reshape(1, _HID) for g in (gas0, gas1, gas2)]
    gad = [g.reshape(1, _HID) for g in (gad0, gad1, gad2)]
    gbr = [r2(gb0), r2(gb1), r2(gb2)]
    lng, lnb = r2(ln_g), r2(ln_b)

    W0s, b0s, Fs, cs = pl.pallas_call(
        _prep_body,
        out_shape=[
            jax.ShapeDtypeStruct((2, D, _HID), f32),
            jax.ShapeDtypeStruct((2, 1, _HID), f32),
            jax.ShapeDtypeStruct((4, D, _HID), f32),
            jax.ShapeDtypeStruct((4, 1, _HID), f32),
        ],
    )(gW0, gW1, gW2, gbr[0], gbr[1], gbr[2],
      W_mouth, W_left_eye, W_right_eye, W_audio, bm, bl, br, ba)

    BT = 1024
    NT = T // BT
    vec_bs = pl.BlockSpec((1, _HID), lambda i: (0, 0))

    s_active = pl.pallas_call(
        _active_body,
        grid=(NT,),
        in_specs=[
            pl.BlockSpec((3, BT, D), lambda i: (0, i, 0)),
            pl.BlockSpec((1, BT, A), lambda i: (0, i, 0)),
            pl.BlockSpec((2, D, _HID), lambda i: (0, 0, 0)),
            pl.BlockSpec((2, 1, _HID), lambda i: (0, 0, 0)),
            pl.BlockSpec((_HID, _HID), lambda i: (0, 0)),
            pl.BlockSpec((_HID, _HID), lambda i: (0, 0)),
            vec_bs, vec_bs, vec_bs, vec_bs, vec_bs, vec_bs,
            vec_bs, vec_bs, vec_bs, vec_bs, vec_bs,
        ],
        out_specs=pl.BlockSpec((1, _HID), lambda i: (0, 0)),
        out_shape=jax.ShapeDtypeStruct((1, _HID), f32),
        scratch_shapes=[
            pltpu.VMEM((3, _HID), f32),
            pltpu.VMEM((3, _HID), f32),
            pltpu.VMEM((3, _HID), f32),
        ],
    )(region_mouth, audio_embeddings, W0s, b0s, gW1, gW2,
      gas[0], gad[0], gas[1], gad[1], gas[2], gad[2],
      gbr[0], gbr[1], gbr[2], lng, lnb)

    def passive_sum(raw, group):
        n = raw.shape[0]
        tiles = n // T
        return pl.pallas_call(
            _passive_body,
            grid=(tiles,),
            in_specs=[
                pl.BlockSpec((T, D), lambda i: (i, 0)),
                pl.BlockSpec((1, D, _HID), lambda i, g=group: (g, 0, 0)),
                pl.BlockSpec((1, 1, _HID), lambda i, g=group: (g, 0, 0)),
                vec_bs, vec_bs,
            ],
            out_specs=pl.BlockSpec((1, _HID), lambda i: (0, 0)),
            out_shape=jax.ShapeDtypeStruct((1, _HID), f32),
        )(raw, Fs, cs, lng, lnb)

    s_m = passive_sum(region_mouth[3], 0)
    s_l = passive_sum(region_left_eye.reshape(B * T, D), 1)
    s_r = passive_sum(region_right_eye.reshape(B * T, D), 2)
    s_a = passive_sum(audio_embeddings[1:].reshape((B - 1) * T_a, A), 3)

    total = s_active + s_m + s_l + s_r + s_a
    return total / jnp.float32(N_total)
